# baseline (speedup 1.0000x reference)
import functools

import jax
import jax.numpy as jnp
from jax import lax
from jax.experimental import pallas as pl
from jax.experimental.pallas import tpu as pltpu
from jax.experimental.pallas import tpu_sc as plsc

N = 10000
E = 320000
D = 128
H = 128
C = 10

NC = 2
NS = 16
NW = NC * NS
CHUNK = 128
N_PAD = 10240
ROWS_PER_TILE = N_PAD // NS
NBUF = 2
CHUNKS = 80
E_PAD = NW * CHUNKS * CHUNK
BL = 1024
GRID = N_PAD // BL


@functools.cache
def _make_sc_conv():
  mesh = plsc.VectorSubcoreMesh(core_axis_name="c", subcore_axis_name="s",
                                num_cores=NC)

  @functools.partial(
      pl.kernel,
      mesh=mesh,
      out_type=jax.ShapeDtypeStruct((NC, N_PAD, H), jnp.float32),
      scratch_types=[
          pltpu.VMEM((CHUNKS, CHUNK), jnp.int32),
          pltpu.VMEM_SHARED((N_PAD, H), jnp.float32),
      ] + [pltpu.VMEM((CHUNK,), jnp.int32) for _ in range(2 * NBUF)]
        + [pltpu.VMEM((CHUNK, H), jnp.float32) for _ in range(NBUF)]
        + [pltpu.SemaphoreType.DMA for _ in range(NBUF)],
  )
  def _sc_conv(u_hbm, pk_hbm, zeros_hbm, out_hbm, pk_v, acc_sh, *rest):
    srcb = rest[0:NBUF]
    dstb = rest[NBUF:2 * NBUF]
    bufs = rest[2 * NBUF:3 * NBUF]
    sems = rest[3 * NBUF:4 * NBUF]
    c = lax.axis_index("c")
    s = lax.axis_index("s")
    w = c * NS + s

    def unpack(ii, b):
        for j in range(CHUNK // 16):
            p = pk_v[ii, pl.ds(j * 16, 16)]
            srcb[b][pl.ds(j * 16, 16)] = p & 0x3FFF
            dstb[b][pl.ds(j * 16, 16)] = lax.shift_right_logical(p, 14)

    pltpu.sync_copy(pk_hbm.at[w], pk_v)
    pltpu.sync_copy(zeros_hbm, acc_sh.at[pl.ds(s * ROWS_PER_TILE, ROWS_PER_TILE)])
    for b in range(NBUF):
        unpack(b, b)
        pltpu.async_copy(u_hbm.at[srcb[b]], bufs[b], sems[b])
    plsc.subcore_barrier()

    def step(k, carry):
        i = k * NBUF
        for b in range(NBUF):
            ii = i + b
            pltpu.make_async_copy(u_hbm.at[srcb[b]], bufs[b], sems[b]).wait()
            pltpu.sync_copy(bufs[b], acc_sh.at[dstb[b]], add=True)
            unpack(ii + NBUF, b)
            pltpu.async_copy(u_hbm.at[srcb[b]], bufs[b], sems[b])
        return carry

    lax.fori_loop(0, CHUNKS // NBUF - 1, step, 0)
    for b in range(NBUF):
        pltpu.make_async_copy(u_hbm.at[srcb[b]], bufs[b], sems[b]).wait()
        pltpu.sync_copy(bufs[b], acc_sh.at[dstb[b]], add=True)
    plsc.subcore_barrier()
    pltpu.sync_copy(acc_sh.at[pl.ds(s * ROWS_PER_TILE, ROWS_PER_TILE)],
                    out_hbm.at[c, pl.ds(s * ROWS_PER_TILE, ROWS_PER_TILE)])

  return _sc_conv


def _rows(i):
    return i * BL + lax.broadcasted_iota(jnp.int32, (BL, 1), 0)


def _deg_dinv(degp_ref):
    deg = degp_ref[0, :, 0:1] + degp_ref[1, :, 0:1]
    dinv = 1.0 / jnp.sqrt(jnp.maximum(deg, 1.0))
    return deg, dinv


def _encode_body(x_ref, degp_ref, wx_ref, ws_ref, b_ref, u_ref):
    i = pl.program_id(0)
    deg, dinv = _deg_dinv(degp_ref)
    struct = jnp.log(deg + 1.0)
    h = (jnp.dot(x_ref[...], wx_ref[...], preferred_element_type=jnp.float32)
         + struct * ws_ref[...] + b_ref[...])
    u_ref[...] = jnp.where(_rows(i) < N, h * dinv, 0.0)


def _conv_body(sp_ref, degp_ref, w_ref, b_ref, u_ref, h_ref):
    i = pl.program_id(0)
    _, dinv = _deg_dinv(degp_ref)
    agg = (sp_ref[0] + sp_ref[1]) * dinv
    h = jnp.maximum(
        jnp.dot(agg, w_ref[...], preferred_element_type=jnp.float32) + b_ref[...],
        0.0)
    h = jnp.where(_rows(i) < N, h, 0.0)
    h_ref[...] = h
    u_ref[...] = h * dinv


def _final_body(h_ref, g_ref, be_ref, wh_ref, bh_ref, out_ref, acc_ref):
    i = pl.program_id(0)

    @pl.when(i == 0)
    def _():
        acc_ref[...] = jnp.zeros_like(acc_ref)

    h = h_ref[...]
    mu = jnp.mean(h, axis=1, keepdims=True)
    var = jnp.mean((h - mu) ** 2, axis=1, keepdims=True)
    hn = (h - mu) / jnp.sqrt(var + 1e-5) * g_ref[...] + be_ref[...]
    hn = jnp.where(_rows(i) < N, hn, 0.0)
    acc_ref[...] += jnp.sum(hn, axis=0, keepdims=True)

    @pl.when(i == GRID - 1)
    def _():
        g = acc_ref[...] * (1.0 / N)
        out_ref[...] = (jnp.dot(g, wh_ref[...], preferred_element_type=jnp.float32)
                        + bh_ref[...])


_row_spec = pl.BlockSpec((BL, H), lambda i: (i, 0))
_degp_spec = pl.BlockSpec((NC, BL, H), lambda i: (0, i, 0))
_sp_spec = pl.BlockSpec((NC, BL, H), lambda i: (0, i, 0))
_w_spec = pl.BlockSpec((H, H), lambda i: (0, 0))
_b_spec = pl.BlockSpec((1, H), lambda i: (0, 0))

_encode_call = pl.pallas_call(
    _encode_body,
    grid=(GRID,),
    in_specs=[_row_spec, _degp_spec, _w_spec, _b_spec, _b_spec],
    out_specs=_row_spec,
    out_shape=jax.ShapeDtypeStruct((N_PAD, H), jnp.float32),
)

_conv_call = pl.pallas_call(
    _conv_body,
    grid=(GRID,),
    in_specs=[_sp_spec, _degp_spec, _w_spec, _b_spec],
    out_specs=[_row_spec, _row_spec],
    out_shape=[jax.ShapeDtypeStruct((N_PAD, H), jnp.float32),
               jax.ShapeDtypeStruct((N_PAD, H), jnp.float32)],
)

_final_call = pl.pallas_call(
    _final_body,
    grid=(GRID,),
    in_specs=[_row_spec, _b_spec, _b_spec, _w_spec, _b_spec],
    out_specs=pl.BlockSpec((1, H), lambda i: (0, 0)),
    out_shape=jax.ShapeDtypeStruct((1, H), jnp.float32),
    scratch_shapes=[pltpu.VMEM((1, H), jnp.float32)],
)


def kernel(x, edge_index, W_enc, b_enc, W_c0, b_c0, W_c1, b_c1, W_c2, b_c2,
           gamma, beta, W_head, b_head):
    src = edge_index[0].astype(jnp.int32)
    dst = edge_index[1].astype(jnp.int32)
    pad_e = E_PAD - E
    packed = src | (dst << 14)
    pk_r = jnp.pad(packed, (0, pad_e),
                   constant_values=(N_PAD - 1) | ((N_PAD - 1) << 14)).reshape(
        NW, CHUNKS, CHUNK)
    x_p = jnp.pad(x, ((0, N_PAD - N), (0, 0)))
    zerosH = jnp.zeros((ROWS_PER_TILE, H), jnp.float32)
    row_ids = lax.broadcasted_iota(jnp.int32, (N_PAD, 1), 0)
    ones_mat = jnp.where(row_ids < N, 1.0, 0.0) * jnp.ones((1, H), jnp.float32)
    wx = W_enc[:D]
    ws = W_enc[D:D + 1]
    b_enc2 = b_enc.reshape(1, H)
    gamma2 = gamma.reshape(1, H)
    beta2 = beta.reshape(1, H)
    wh = jnp.pad(W_head, ((0, 0), (0, H - C)))
    bh = jnp.pad(b_head, (0, H - C)).reshape(1, H)

    degp = _make_sc_conv()(ones_mat, pk_r, zerosH)

    u = _encode_call(x_p, degp, wx, ws, b_enc2)

    for W, b in ((W_c0, b_c0), (W_c1, b_c1), (W_c2, b_c2)):
        sp = _make_sc_conv()(u, pk_r, zerosH)
        u, h = _conv_call(sp, degp, W, b.reshape(1, H))

    out = _final_call(h, gamma2, beta2, wh, bh)
    return out[:, :C]


# speedup vs baseline: 1.5377x; 1.5377x over previous
import functools

import jax
import jax.numpy as jnp
from jax import lax
from jax.experimental import pallas as pl
from jax.experimental.pallas import tpu as pltpu
from jax.experimental.pallas import tpu_sc as plsc

N = 10000
E = 320000
D = 128
H = 128
C = 10

NC = 2
NS = 16
H2 = H // NC
N_PAD = 10240
ROWS_PER_TILE = N_PAD // NS
NBUF = 2
BIG = 512
CHUNKS = 40
E_PAD = NS * CHUNKS * BIG
BL = 1024
GRID = N_PAD // BL


@functools.cache
def _make_sc_conv():
  mesh = plsc.VectorSubcoreMesh(core_axis_name="c", subcore_axis_name="s",
                                num_cores=NC)

  @functools.partial(
      pl.kernel,
      mesh=mesh,
      compiler_params=pltpu.CompilerParams(use_tc_tiling_on_sc=False),
      out_type=jax.ShapeDtypeStruct((NC, N_PAD, H2), jnp.float32),
      scratch_types=[
          pltpu.VMEM((CHUNKS, BIG), jnp.int32),
          pltpu.VMEM_SHARED((N_PAD, H2), jnp.float32),
      ] + [pltpu.VMEM((BIG,), jnp.int32) for _ in range(2 * NBUF)]
        + [pltpu.VMEM((BIG, H2), jnp.float32) for _ in range(NBUF)]
        + [pltpu.SemaphoreType.DMA for _ in range(NBUF)],
  )
  def _sc_conv(u_hbm, pk_hbm, zeros_hbm, out_hbm, pk_v, acc_sh, *rest):
    srcb = rest[0:NBUF]
    dstb = rest[NBUF:2 * NBUF]
    bufs = rest[2 * NBUF:3 * NBUF]
    sems = rest[3 * NBUF:4 * NBUF]
    c = lax.axis_index("c")
    s = lax.axis_index("s")
    u_c = u_hbm.at[c]

    def unpack(ii, b):
        for j in range(BIG // 16):
            p = pk_v[ii, pl.ds(j * 16, 16)]
            srcb[b][pl.ds(j * 16, 16)] = p & 0x3FFF
            dstb[b][pl.ds(j * 16, 16)] = lax.shift_right_logical(p, 14)

    pltpu.sync_copy(pk_hbm.at[s], pk_v)
    pltpu.sync_copy(zeros_hbm, acc_sh.at[pl.ds(s * ROWS_PER_TILE, ROWS_PER_TILE)])
    for b in range(NBUF):
        unpack(b, b)
        pltpu.async_copy(u_c.at[srcb[b]], bufs[b], sems[b])
    plsc.subcore_barrier()

    def step(k, carry):
        i = k * NBUF
        for b in range(NBUF):
            ii = i + b
            pltpu.make_async_copy(u_c.at[srcb[b]], bufs[b], sems[b]).wait()
            pltpu.sync_copy(bufs[b], acc_sh.at[dstb[b]], add=True)
            unpack(ii + NBUF, b)
            pltpu.async_copy(u_c.at[srcb[b]], bufs[b], sems[b])
        return carry

    lax.fori_loop(0, CHUNKS // NBUF - 1, step, 0)
    for b in range(NBUF):
        pltpu.make_async_copy(u_c.at[srcb[b]], bufs[b], sems[b]).wait()
        pltpu.sync_copy(bufs[b], acc_sh.at[dstb[b]], add=True)
    plsc.subcore_barrier()
    pltpu.sync_copy(acc_sh.at[pl.ds(s * ROWS_PER_TILE, ROWS_PER_TILE)],
                    out_hbm.at[c, pl.ds(s * ROWS_PER_TILE, ROWS_PER_TILE)])

  return _sc_conv


def _rows(i):
    return i * BL + lax.broadcasted_iota(jnp.int32, (BL, 1), 0)


def _deg_dinv(degp_ref):
    deg = degp_ref[0, :, 0:1]
    dinv = 1.0 / jnp.sqrt(jnp.maximum(deg, 1.0))
    return deg, dinv


def _split_store(u_ref, u):
    u_ref[0] = u[:, :H2]
    u_ref[1] = u[:, H2:]


def _encode_body(x_ref, degp_ref, wx_ref, ws_ref, b_ref, u_ref):
    i = pl.program_id(0)
    deg, dinv = _deg_dinv(degp_ref)
    struct = jnp.log(deg + 1.0)
    h = (jnp.dot(x_ref[...], wx_ref[...], preferred_element_type=jnp.float32)
         + struct * ws_ref[...] + b_ref[...])
    _split_store(u_ref, jnp.where(_rows(i) < N, h * dinv, 0.0))


def _conv_body(sp_ref, degp_ref, w_ref, b_ref, u_ref, h_ref):
    i = pl.program_id(0)
    _, dinv = _deg_dinv(degp_ref)
    agg = jnp.concatenate([sp_ref[0], sp_ref[1]], axis=1) * dinv
    h = jnp.maximum(
        jnp.dot(agg, w_ref[...], preferred_element_type=jnp.float32) + b_ref[...],
        0.0)
    h = jnp.where(_rows(i) < N, h, 0.0)
    h_ref[...] = h
    _split_store(u_ref, h * dinv)


def _final_body(h_ref, g_ref, be_ref, wh_ref, bh_ref, out_ref, acc_ref):
    i = pl.program_id(0)

    @pl.when(i == 0)
    def _():
        acc_ref[...] = jnp.zeros_like(acc_ref)

    h = h_ref[...]
    mu = jnp.mean(h, axis=1, keepdims=True)
    var = jnp.mean((h - mu) ** 2, axis=1, keepdims=True)
    hn = (h - mu) / jnp.sqrt(var + 1e-5) * g_ref[...] + be_ref[...]
    hn = jnp.where(_rows(i) < N, hn, 0.0)
    acc_ref[...] += jnp.sum(hn, axis=0, keepdims=True)

    @pl.when(i == GRID - 1)
    def _():
        g = acc_ref[...] * (1.0 / N)
        out_ref[...] = (jnp.dot(g, wh_ref[...], preferred_element_type=jnp.float32)
                        + bh_ref[...])


_row_spec = pl.BlockSpec((BL, H), lambda i: (i, 0))
_half_spec = pl.BlockSpec((NC, BL, H2), lambda i: (0, i, 0))
_w_spec = pl.BlockSpec((H, H), lambda i: (0, 0))
_b_spec = pl.BlockSpec((1, H), lambda i: (0, 0))

_half_shape = jax.ShapeDtypeStruct((NC, N_PAD, H2), jnp.float32)

_encode_call = pl.pallas_call(
    _encode_body,
    grid=(GRID,),
    in_specs=[_row_spec, _half_spec, _w_spec, _b_spec, _b_spec],
    out_specs=_half_spec,
    out_shape=_half_shape,
)

_conv_call = pl.pallas_call(
    _conv_body,
    grid=(GRID,),
    in_specs=[_half_spec, _half_spec, _w_spec, _b_spec],
    out_specs=[_half_spec, _row_spec],
    out_shape=[_half_shape,
               jax.ShapeDtypeStruct((N_PAD, H), jnp.float32)],
)

_final_call = pl.pallas_call(
    _final_body,
    grid=(GRID,),
    in_specs=[_row_spec, _b_spec, _b_spec, _w_spec, _b_spec],
    out_specs=pl.BlockSpec((1, H), lambda i: (0, 0)),
    out_shape=jax.ShapeDtypeStruct((1, H), jnp.float32),
    scratch_shapes=[pltpu.VMEM((1, H), jnp.float32)],
)


def kernel(x, edge_index, W_enc, b_enc, W_c0, b_c0, W_c1, b_c1, W_c2, b_c2,
           gamma, beta, W_head, b_head):
    src = edge_index[0].astype(jnp.int32)
    dst = edge_index[1].astype(jnp.int32)
    pad_e = E_PAD - E
    packed = src | (dst << 14)
    pk_r = jnp.pad(packed, (0, pad_e),
                   constant_values=(N_PAD - 1) | ((N_PAD - 1) << 14)).reshape(
        NS, CHUNKS, BIG)
    x_p = jnp.pad(x, ((0, N_PAD - N), (0, 0)))
    zerosH = jnp.zeros((ROWS_PER_TILE, H2), jnp.float32)
    row_ids = lax.broadcasted_iota(jnp.int32, (N_PAD, 1), 0)
    ones_mat = jnp.broadcast_to(
        jnp.where(row_ids < N, 1.0, 0.0)[None], (NC, N_PAD, H2))
    wx = W_enc[:D]
    ws = W_enc[D:D + 1]
    b_enc2 = b_enc.reshape(1, H)
    gamma2 = gamma.reshape(1, H)
    beta2 = beta.reshape(1, H)
    wh = jnp.pad(W_head, ((0, 0), (0, H - C)))
    bh = jnp.pad(b_head, (0, H - C)).reshape(1, H)

    degp = _make_sc_conv()(ones_mat, pk_r, zerosH)

    u = _encode_call(x_p, degp, wx, ws, b_enc2)

    for W, b in ((W_c0, b_c0), (W_c1, b_c1), (W_c2, b_c2)):
        sp = _make_sc_conv()(u, pk_r, zerosH)
        u, h = _conv_call(sp, degp, W, b.reshape(1, H))

    out = _final_call(h, gamma2, beta2, wh, bh)
    return out[:, :C]


# speedup vs baseline: 1.7174x; 1.1169x over previous
import functools

import jax
import jax.numpy as jnp
from jax import lax
from jax.experimental import pallas as pl
from jax.experimental.pallas import tpu as pltpu
from jax.experimental.pallas import tpu_sc as plsc

N = 10000
E = 320000
D = 128
H = 128
C = 10

NC = 2
NS = 16
H2 = H // NC
N_PAD = 10240
ROWS_PER_TILE = N_PAD // NS
NBUF = 2
BIG = 512
CHUNKS = 40
E_PAD = NS * CHUNKS * BIG
BL = 1024
GRID = N_PAD // BL


@functools.cache
def _make_sc_conv():
  mesh = plsc.VectorSubcoreMesh(core_axis_name="c", subcore_axis_name="s",
                                num_cores=NC)

  @functools.partial(
      pl.kernel,
      mesh=mesh,
      compiler_params=pltpu.CompilerParams(use_tc_tiling_on_sc=False),
      out_type=jax.ShapeDtypeStruct((NC, N_PAD, H2), jnp.float32),
      scratch_types=[
          pltpu.VMEM((CHUNKS, BIG), jnp.int32),
          pltpu.VMEM_SHARED((N_PAD, H2), jnp.float32),
      ] + [pltpu.VMEM((BIG,), jnp.int32) for _ in range(2 * NBUF)]
        + [pltpu.VMEM((BIG, H2), jnp.float32) for _ in range(NBUF)]
        + [pltpu.SemaphoreType.DMA for _ in range(NBUF)],
  )
  def _sc_conv(u_hbm, pk_hbm, zeros_hbm, out_hbm, pk_v, acc_sh, *rest):
    srcb = rest[0:NBUF]
    dstb = rest[NBUF:2 * NBUF]
    bufs = rest[2 * NBUF:3 * NBUF]
    sems = rest[3 * NBUF:4 * NBUF]
    c = lax.axis_index("c")
    s = lax.axis_index("s")
    u_c = u_hbm.at[c]

    def unpack(ii, b):
        for j in range(BIG // 16):
            p = pk_v[ii, pl.ds(j * 16, 16)]
            srcb[b][pl.ds(j * 16, 16)] = p & 0x3FFF
            dstb[b][pl.ds(j * 16, 16)] = lax.shift_right_logical(p, 14)

    pltpu.sync_copy(pk_hbm.at[s], pk_v)
    pltpu.sync_copy(zeros_hbm, acc_sh.at[pl.ds(s * ROWS_PER_TILE, ROWS_PER_TILE)])
    for b in range(NBUF):
        unpack(b, b)
        pltpu.async_copy(u_c.at[srcb[b]], bufs[b], sems[b])
    plsc.subcore_barrier()

    def step(k, carry):
        i = k * NBUF
        for b in range(NBUF):
            ii = i + b
            pltpu.make_async_copy(u_c.at[srcb[b]], bufs[b], sems[b]).wait()
            pltpu.sync_copy(bufs[b], acc_sh.at[dstb[b]], add=True)
            unpack(ii + NBUF, b)
            pltpu.async_copy(u_c.at[srcb[b]], bufs[b], sems[b])
        return carry

    lax.fori_loop(0, CHUNKS // NBUF - 1, step, 0)
    for b in range(NBUF):
        pltpu.make_async_copy(u_c.at[srcb[b]], bufs[b], sems[b]).wait()
        pltpu.sync_copy(bufs[b], acc_sh.at[dstb[b]], add=True)
    plsc.subcore_barrier()
    pltpu.sync_copy(acc_sh.at[pl.ds(s * ROWS_PER_TILE, ROWS_PER_TILE)],
                    out_hbm.at[c, pl.ds(s * ROWS_PER_TILE, ROWS_PER_TILE)])

  return _sc_conv


@functools.cache
def _make_sc_deg():
  mesh = plsc.VectorSubcoreMesh(core_axis_name="c", subcore_axis_name="s",
                                num_cores=NC)

  @functools.partial(
      pl.kernel,
      mesh=mesh,
      compiler_params=pltpu.CompilerParams(use_tc_tiling_on_sc=False),
      out_type=jax.ShapeDtypeStruct((NC, N_PAD, 16), jnp.float32),
      scratch_types=[
          pltpu.VMEM((CHUNKS // NC, BIG), jnp.int32),
          pltpu.VMEM((BIG,), jnp.int32),
          pltpu.VMEM((BIG, 16), jnp.float32),
          pltpu.VMEM_SHARED((N_PAD, 16), jnp.float32),
      ],
  )
  def _sc_deg(pk_hbm, ones_hbm, zeros_hbm, out_hbm, pk_v, dstb, ones_v, acc_sh):
    c = lax.axis_index("c")
    s = lax.axis_index("s")
    half = CHUNKS // NC
    pltpu.sync_copy(pk_hbm.at[s, pl.ds(c * half, half)], pk_v)
    pltpu.sync_copy(ones_hbm, ones_v)
    pltpu.sync_copy(zeros_hbm, acc_sh.at[pl.ds(s * ROWS_PER_TILE, ROWS_PER_TILE)])
    plsc.subcore_barrier()

    def step(ii, carry):
        for j in range(BIG // 16):
            p = pk_v[ii, pl.ds(j * 16, 16)]
            dstb[pl.ds(j * 16, 16)] = lax.shift_right_logical(p, 14)
        pltpu.sync_copy(ones_v, acc_sh.at[dstb], add=True)
        return carry

    lax.fori_loop(0, half, step, 0)
    plsc.subcore_barrier()
    pltpu.sync_copy(acc_sh.at[pl.ds(s * ROWS_PER_TILE, ROWS_PER_TILE)],
                    out_hbm.at[c, pl.ds(s * ROWS_PER_TILE, ROWS_PER_TILE)])

  return _sc_deg


def _rows(i):
    return i * BL + lax.broadcasted_iota(jnp.int32, (BL, 1), 0)


def _deg_dinv(degp_ref):
    deg = degp_ref[0, :, 0:1] + degp_ref[1, :, 0:1]
    dinv = 1.0 / jnp.sqrt(jnp.maximum(deg, 1.0))
    return deg, dinv


def _split_store(u_ref, u):
    u_ref[0] = u[:, :H2]
    u_ref[1] = u[:, H2:]


def _encode_body(x_ref, degp_ref, wx_ref, ws_ref, b_ref, u_ref):
    i = pl.program_id(0)
    deg, dinv = _deg_dinv(degp_ref)
    struct = jnp.log(deg + 1.0)
    h = (jnp.dot(x_ref[...], wx_ref[...], preferred_element_type=jnp.float32)
         + struct * ws_ref[...] + b_ref[...])
    _split_store(u_ref, jnp.where(_rows(i) < N, h * dinv, 0.0))


def _conv_body(sp_ref, degp_ref, w_ref, b_ref, u_ref, h_ref):
    i = pl.program_id(0)
    _, dinv = _deg_dinv(degp_ref)
    agg = jnp.concatenate([sp_ref[0], sp_ref[1]], axis=1) * dinv
    h = jnp.maximum(
        jnp.dot(agg, w_ref[...], preferred_element_type=jnp.float32) + b_ref[...],
        0.0)
    h = jnp.where(_rows(i) < N, h, 0.0)
    h_ref[...] = h
    _split_store(u_ref, h * dinv)


def _final_body(h_ref, g_ref, be_ref, wh_ref, bh_ref, out_ref, acc_ref):
    i = pl.program_id(0)

    @pl.when(i == 0)
    def _():
        acc_ref[...] = jnp.zeros_like(acc_ref)

    h = h_ref[...]
    mu = jnp.mean(h, axis=1, keepdims=True)
    var = jnp.mean((h - mu) ** 2, axis=1, keepdims=True)
    hn = (h - mu) / jnp.sqrt(var + 1e-5) * g_ref[...] + be_ref[...]
    hn = jnp.where(_rows(i) < N, hn, 0.0)
    acc_ref[...] += jnp.sum(hn, axis=0, keepdims=True)

    @pl.when(i == GRID - 1)
    def _():
        g = acc_ref[...] * (1.0 / N)
        out_ref[...] = (jnp.dot(g, wh_ref[...], preferred_element_type=jnp.float32)
                        + bh_ref[...])


_row_spec = pl.BlockSpec((BL, H), lambda i: (i, 0))
_half_spec = pl.BlockSpec((NC, BL, H2), lambda i: (0, i, 0))
_degp_spec = pl.BlockSpec((NC, BL, 16), lambda i: (0, i, 0))
_w_spec = pl.BlockSpec((H, H), lambda i: (0, 0))
_b_spec = pl.BlockSpec((1, H), lambda i: (0, 0))

_half_shape = jax.ShapeDtypeStruct((NC, N_PAD, H2), jnp.float32)

_encode_call = pl.pallas_call(
    _encode_body,
    grid=(GRID,),
    in_specs=[_row_spec, _degp_spec, _w_spec, _b_spec, _b_spec],
    out_specs=_half_spec,
    out_shape=_half_shape,
)

_conv_call = pl.pallas_call(
    _conv_body,
    grid=(GRID,),
    in_specs=[_half_spec, _degp_spec, _w_spec, _b_spec],
    out_specs=[_half_spec, _row_spec],
    out_shape=[_half_shape,
               jax.ShapeDtypeStruct((N_PAD, H), jnp.float32)],
)

_final_call = pl.pallas_call(
    _final_body,
    grid=(GRID,),
    in_specs=[_row_spec, _b_spec, _b_spec, _w_spec, _b_spec],
    out_specs=pl.BlockSpec((1, H), lambda i: (0, 0)),
    out_shape=jax.ShapeDtypeStruct((1, H), jnp.float32),
    scratch_shapes=[pltpu.VMEM((1, H), jnp.float32)],
)


def kernel(x, edge_index, W_enc, b_enc, W_c0, b_c0, W_c1, b_c1, W_c2, b_c2,
           gamma, beta, W_head, b_head):
    src = edge_index[0].astype(jnp.int32)
    dst = edge_index[1].astype(jnp.int32)
    pad_e = E_PAD - E
    packed = src | (dst << 14)
    pk_r = jnp.pad(packed, (0, pad_e),
                   constant_values=(N_PAD - 1) | ((N_PAD - 1) << 14)).reshape(
        NS, CHUNKS, BIG)
    x_p = jnp.pad(x, ((0, N_PAD - N), (0, 0)))
    zerosH = jnp.zeros((ROWS_PER_TILE, H2), jnp.float32)
    zeros16 = jnp.zeros((ROWS_PER_TILE, 16), jnp.float32)
    ones16 = jnp.ones((BIG, 16), jnp.float32)
    wx = W_enc[:D]
    ws = W_enc[D:D + 1]
    b_enc2 = b_enc.reshape(1, H)
    gamma2 = gamma.reshape(1, H)
    beta2 = beta.reshape(1, H)
    wh = jnp.pad(W_head, ((0, 0), (0, H - C)))
    bh = jnp.pad(b_head, (0, H - C)).reshape(1, H)

    degp = _make_sc_deg()(pk_r, ones16, zeros16)

    u = _encode_call(x_p, degp, wx, ws, b_enc2)

    for W, b in ((W_c0, b_c0), (W_c1, b_c1), (W_c2, b_c2)):
        sp = _make_sc_conv()(u, pk_r, zerosH)
        u, h = _conv_call(sp, degp, W, b.reshape(1, H))

    out = _final_call(h, gamma2, beta2, wh, bh)
    return out[:, :C]


# speedup vs baseline: 1.7247x; 1.0042x over previous
import functools

import jax
import jax.numpy as jnp
from jax import lax
from jax.experimental import pallas as pl
from jax.experimental.pallas import tpu as pltpu
from jax.experimental.pallas import tpu_sc as plsc

N = 10000
E = 320000
D = 128
H = 128
C = 10

NC = 2
NS = 16
H2 = H // NC
N_PAD = 10240
ROWS_PER_TILE = N_PAD // NS
NBUF = 2
BIG = 512
CHUNKS = 40
E_PAD = NS * CHUNKS * BIG
BL = 1024
GRID = N_PAD // BL


@functools.cache
def _make_sc_conv():
  mesh = plsc.VectorSubcoreMesh(core_axis_name="c", subcore_axis_name="s",
                                num_cores=NC)

  @functools.partial(
      pl.kernel,
      mesh=mesh,
      compiler_params=pltpu.CompilerParams(use_tc_tiling_on_sc=False),
      out_type=jax.ShapeDtypeStruct((NC, N_PAD, H2), jnp.float32),
      scratch_types=[
          pltpu.VMEM((CHUNKS, BIG), jnp.int32),
          pltpu.VMEM_SHARED((N_PAD, H2), jnp.float32),
      ] + [pltpu.VMEM((BIG,), jnp.int32) for _ in range(2 * NBUF)]
        + [pltpu.VMEM((BIG, H2), jnp.float32) for _ in range(NBUF)]
        + [pltpu.SemaphoreType.DMA for _ in range(2 * NBUF)],
  )
  def _sc_conv(u_hbm, pk_hbm, zeros_hbm, out_hbm, pk_v, acc_sh, *rest):
    srcb = rest[0:NBUF]
    dstb = rest[NBUF:2 * NBUF]
    bufs = rest[2 * NBUF:3 * NBUF]
    gsem = rest[3 * NBUF:4 * NBUF]
    ssem = rest[4 * NBUF:5 * NBUF]
    c = lax.axis_index("c")
    s = lax.axis_index("s")
    u_c = u_hbm.at[c]

    def unpack(ii, b):
        for j in range(BIG // 16):
            p = pk_v[ii, pl.ds(j * 16, 16)]
            srcb[b][pl.ds(j * 16, 16)] = p & 0x3FFF
            dstb[b][pl.ds(j * 16, 16)] = lax.shift_right_logical(p, 14)

    def fire_gather(b):
        pltpu.async_copy(u_c.at[srcb[b]], bufs[b], gsem[b])

    def wait_gather(b):
        pltpu.make_async_copy(u_c.at[srcb[b]], bufs[b], gsem[b]).wait()

    def fire_scatter(b):
        pltpu.async_copy(bufs[b], acc_sh.at[dstb[b]], ssem[b], add=True)

    def wait_scatter(b):
        pltpu.make_async_copy(bufs[b], acc_sh.at[dstb[b]], ssem[b]).wait()

    pltpu.sync_copy(pk_hbm.at[s], pk_v)
    pltpu.sync_copy(zeros_hbm, acc_sh.at[pl.ds(s * ROWS_PER_TILE, ROWS_PER_TILE)])
    unpack(0, 0)
    fire_gather(0)
    plsc.subcore_barrier()
    unpack(1, 1)
    fire_gather(1)
    wait_gather(0)
    fire_scatter(0)

    def visit(ii, b):
        ob = 1 - b
        wait_scatter(ob)
        unpack(ii + 1, ob)
        fire_gather(ob)
        wait_gather(b)
        fire_scatter(b)

    def step(k, carry):
        visit(2 * k + 1, 1)
        visit(2 * k + 2, 0)
        return carry

    lax.fori_loop(0, (CHUNKS - 2) // 2, step, 0)
    wait_gather(1)
    fire_scatter(1)
    wait_scatter(0)
    wait_scatter(1)
    plsc.subcore_barrier()
    pltpu.sync_copy(acc_sh.at[pl.ds(s * ROWS_PER_TILE, ROWS_PER_TILE)],
                    out_hbm.at[c, pl.ds(s * ROWS_PER_TILE, ROWS_PER_TILE)])

  return _sc_conv


@functools.cache
def _make_sc_deg():
  mesh = plsc.VectorSubcoreMesh(core_axis_name="c", subcore_axis_name="s",
                                num_cores=NC)

  @functools.partial(
      pl.kernel,
      mesh=mesh,
      compiler_params=pltpu.CompilerParams(use_tc_tiling_on_sc=False),
      out_type=jax.ShapeDtypeStruct((NC, N_PAD, 16), jnp.float32),
      scratch_types=[
          pltpu.VMEM((CHUNKS // NC, BIG), jnp.int32),
          pltpu.VMEM((BIG,), jnp.int32),
          pltpu.VMEM((BIG, 16), jnp.float32),
          pltpu.VMEM_SHARED((N_PAD, 16), jnp.float32),
      ],
  )
  def _sc_deg(pk_hbm, ones_hbm, zeros_hbm, out_hbm, pk_v, dstb, ones_v, acc_sh):
    c = lax.axis_index("c")
    s = lax.axis_index("s")
    half = CHUNKS // NC
    pltpu.sync_copy(pk_hbm.at[s, pl.ds(c * half, half)], pk_v)
    pltpu.sync_copy(ones_hbm, ones_v)
    pltpu.sync_copy(zeros_hbm, acc_sh.at[pl.ds(s * ROWS_PER_TILE, ROWS_PER_TILE)])
    plsc.subcore_barrier()

    def step(ii, carry):
        for j in range(BIG // 16):
            p = pk_v[ii, pl.ds(j * 16, 16)]
            dstb[pl.ds(j * 16, 16)] = lax.shift_right_logical(p, 14)
        pltpu.sync_copy(ones_v, acc_sh.at[dstb], add=True)
        return carry

    lax.fori_loop(0, half, step, 0)
    plsc.subcore_barrier()
    pltpu.sync_copy(acc_sh.at[pl.ds(s * ROWS_PER_TILE, ROWS_PER_TILE)],
                    out_hbm.at[c, pl.ds(s * ROWS_PER_TILE, ROWS_PER_TILE)])

  return _sc_deg


def _rows(i):
    return i * BL + lax.broadcasted_iota(jnp.int32, (BL, 1), 0)


def _deg_dinv(degp_ref):
    deg = degp_ref[0, :, 0:1] + degp_ref[1, :, 0:1]
    dinv = 1.0 / jnp.sqrt(jnp.maximum(deg, 1.0))
    return deg, dinv


def _split_store(u_ref, u):
    u_ref[0] = u[:, :H2]
    u_ref[1] = u[:, H2:]


def _encode_body(x_ref, degp_ref, wx_ref, ws_ref, b_ref, u_ref):
    i = pl.program_id(0)
    deg, dinv = _deg_dinv(degp_ref)
    struct = jnp.log(deg + 1.0)
    h = (jnp.dot(x_ref[...], wx_ref[...], preferred_element_type=jnp.float32)
         + struct * ws_ref[...] + b_ref[...])
    _split_store(u_ref, jnp.where(_rows(i) < N, h * dinv, 0.0))


def _conv_body(sp_ref, degp_ref, w_ref, b_ref, u_ref, h_ref):
    i = pl.program_id(0)
    _, dinv = _deg_dinv(degp_ref)
    agg = jnp.concatenate([sp_ref[0], sp_ref[1]], axis=1) * dinv
    h = jnp.maximum(
        jnp.dot(agg, w_ref[...], preferred_element_type=jnp.float32) + b_ref[...],
        0.0)
    h = jnp.where(_rows(i) < N, h, 0.0)
    h_ref[...] = h
    _split_store(u_ref, h * dinv)


def _final_body(h_ref, g_ref, be_ref, wh_ref, bh_ref, out_ref, acc_ref):
    i = pl.program_id(0)

    @pl.when(i == 0)
    def _():
        acc_ref[...] = jnp.zeros_like(acc_ref)

    h = h_ref[...]
    mu = jnp.mean(h, axis=1, keepdims=True)
    var = jnp.mean((h - mu) ** 2, axis=1, keepdims=True)
    hn = (h - mu) / jnp.sqrt(var + 1e-5) * g_ref[...] + be_ref[...]
    hn = jnp.where(_rows(i) < N, hn, 0.0)
    acc_ref[...] += jnp.sum(hn, axis=0, keepdims=True)

    @pl.when(i == GRID - 1)
    def _():
        g = acc_ref[...] * (1.0 / N)
        out_ref[...] = (jnp.dot(g, wh_ref[...], preferred_element_type=jnp.float32)
                        + bh_ref[...])


_row_spec = pl.BlockSpec((BL, H), lambda i: (i, 0))
_half_spec = pl.BlockSpec((NC, BL, H2), lambda i: (0, i, 0))
_degp_spec = pl.BlockSpec((NC, BL, 16), lambda i: (0, i, 0))
_w_spec = pl.BlockSpec((H, H), lambda i: (0, 0))
_b_spec = pl.BlockSpec((1, H), lambda i: (0, 0))

_half_shape = jax.ShapeDtypeStruct((NC, N_PAD, H2), jnp.float32)

_encode_call = pl.pallas_call(
    _encode_body,
    grid=(GRID,),
    in_specs=[_row_spec, _degp_spec, _w_spec, _b_spec, _b_spec],
    out_specs=_half_spec,
    out_shape=_half_shape,
)

_conv_call = pl.pallas_call(
    _conv_body,
    grid=(GRID,),
    in_specs=[_half_spec, _degp_spec, _w_spec, _b_spec],
    out_specs=[_half_spec, _row_spec],
    out_shape=[_half_shape,
               jax.ShapeDtypeStruct((N_PAD, H), jnp.float32)],
)

_final_call = pl.pallas_call(
    _final_body,
    grid=(GRID,),
    in_specs=[_row_spec, _b_spec, _b_spec, _w_spec, _b_spec],
    out_specs=pl.BlockSpec((1, H), lambda i: (0, 0)),
    out_shape=jax.ShapeDtypeStruct((1, H), jnp.float32),
    scratch_shapes=[pltpu.VMEM((1, H), jnp.float32)],
)


def kernel(x, edge_index, W_enc, b_enc, W_c0, b_c0, W_c1, b_c1, W_c2, b_c2,
           gamma, beta, W_head, b_head):
    src = edge_index[0].astype(jnp.int32)
    dst = edge_index[1].astype(jnp.int32)
    pad_e = E_PAD - E
    packed = src | (dst << 14)
    pk_r = jnp.pad(packed, (0, pad_e),
                   constant_values=(N_PAD - 1) | ((N_PAD - 1) << 14)).reshape(
        NS, CHUNKS, BIG)
    x_p = jnp.pad(x, ((0, N_PAD - N), (0, 0)))
    zerosH = jnp.zeros((ROWS_PER_TILE, H2), jnp.float32)
    zeros16 = jnp.zeros((ROWS_PER_TILE, 16), jnp.float32)
    ones16 = jnp.ones((BIG, 16), jnp.float32)
    wx = W_enc[:D]
    ws = W_enc[D:D + 1]
    b_enc2 = b_enc.reshape(1, H)
    gamma2 = gamma.reshape(1, H)
    beta2 = beta.reshape(1, H)
    wh = jnp.pad(W_head, ((0, 0), (0, H - C)))
    bh = jnp.pad(b_head, (0, H - C)).reshape(1, H)

    degp = _make_sc_deg()(pk_r, ones16, zeros16)

    u = _encode_call(x_p, degp, wx, ws, b_enc2)

    for W, b in ((W_c0, b_c0), (W_c1, b_c1), (W_c2, b_c2)):
        sp = _make_sc_conv()(u, pk_r, zerosH)
        u, h = _conv_call(sp, degp, W, b.reshape(1, H))

    out = _final_call(h, gamma2, beta2, wh, bh)
    return out[:, :C]
